# baseline (speedup 1.0000x reference)
import functools

import jax
import jax.numpy as jnp
from jax import lax
from jax.experimental import pallas as pl
from jax.experimental.pallas import tpu as pltpu
from jax.experimental.pallas import tpu_sc as plsc

_V = 1000000
_D = 64
_DP = 128
_B = 4096 * 50
_NC = 2
_NS = 16
_NW = _NC * _NS
_B_PER_W = _B // _NW
_CHUNK = 400
_NCHUNK = _B_PER_W // _CHUNK

_TCH = 128
_NFULL = _V // _TCH

_mesh = plsc.VectorSubcoreMesh(core_axis_name="c", subcore_axis_name="s")


@functools.partial(
    pl.kernel,
    out_type=jax.ShapeDtypeStruct((_V, _DP), jnp.float32),
    mesh=_mesh,
    scratch_types=[
        pltpu.VMEM((_D, _TCH), jnp.float32),
        pltpu.VMEM((_D, _TCH), jnp.float32),
        pltpu.VMEM((_TCH, _DP), jnp.float32),
        pltpu.VMEM((_TCH, _DP), jnp.float32),
        pltpu.SemaphoreType.DMA,
        pltpu.SemaphoreType.DMA,
        pltpu.SemaphoreType.DMA,
        pltpu.SemaphoreType.DMA,
    ],
    compiler_params=pltpu.CompilerParams(needs_layout_passes=False),
)
def _transpose(wt_hbm, wtail_hbm, out_hbm, s0, s1, t0, t1, ls0, ls1, ss0, ss1):
    wid = lax.axis_index("s") * _NC + lax.axis_index("c")
    n_mine = (_NFULL - wid + _NW - 1) // _NW
    lane16 = lax.iota(jnp.int32, 16)
    zero16 = jnp.zeros((16,), jnp.int32)

    def load(i, sbuf, lsem):
        base = (wid + i * _NW) * _TCH
        pltpu.async_copy(wt_hbm.at[:, pl.ds(base, _TCH)], sbuf, lsem)

    def load_wait(sbuf, lsem):
        pltpu.make_async_copy(wt_hbm.at[:, pl.ds(0, _TCH)], sbuf, lsem).wait()

    def store(i, tbuf, ssem):
        base = (wid + i * _NW) * _TCH
        pltpu.async_copy(tbuf, out_hbm.at[pl.ds(base, _TCH)], ssem)

    def store_wait(tbuf, ssem):
        pltpu.make_async_copy(
            tbuf, out_hbm.at[pl.ds(0, _TCH)], ssem).wait()

    def transpose_slab(src, dst, nrows):
        def row(d, carry):
            dsplat = zero16 + d
            for a in range(4):
                v = plsc.load_gather(src, [lane16 + a * 16, dsplat])
                plsc.store_scatter(dst, [dsplat, lane16 + a * 16], v)
            return carry
        lax.fori_loop(0, nrows, row, 0, unroll=2)

    load(0, s0, ls0)

    def pair(p, carry):
        i0 = 2 * p
        i1 = i0 + 1

        @pl.when(i0 < n_mine)
        def _():
            @pl.when(i1 < n_mine)
            def _():
                load(i1, s1, ls1)
            load_wait(s0, ls0)
            transpose_slab(s0, t0, _TCH)

            @pl.when(i0 >= 2)
            def _():
                store_wait(t0, ss0)
            store(i0, t0, ss0)

        @pl.when(i1 < n_mine)
        def _():
            @pl.when(i1 + 1 < n_mine)
            def _():
                load(i1 + 1, s0, ls0)
            load_wait(s1, ls1)
            transpose_slab(s1, t1, _TCH)

            @pl.when(i1 >= 2)
            def _():
                store_wait(t1, ss1)
            store(i1, t1, ss1)
        return carry

    lax.fori_loop(0, (n_mine + 1) // 2, pair, 0)

    store_wait(t0, ss0)
    store_wait(t1, ss1)

    @pl.when(wid == _NFULL % _NW)
    def _():
        pltpu.sync_copy(wtail_hbm, t0.at[pl.ds(0, _D)])
        pltpu.sync_copy(t0.at[pl.ds(0, _D)],
                        out_hbm.at[pl.ds(_NFULL * _TCH, _D)])


@functools.partial(
    pl.kernel,
    out_type=jax.ShapeDtypeStruct((_B, _DP), jnp.float32),
    mesh=_mesh,
    scratch_types=[
        pltpu.VMEM((_B_PER_W,), jnp.int32),
        pltpu.VMEM((_CHUNK, _DP), jnp.float32),
        pltpu.VMEM((_CHUNK, _DP), jnp.float32),
        pltpu.SemaphoreType.DMA,
        pltpu.SemaphoreType.DMA,
        pltpu.SemaphoreType.DMA,
        pltpu.SemaphoreType.DMA,
    ],
)
def _gather(seq_hbm, table_hbm, out_hbm, idx_v, rows0, rows1,
            gsem0, gsem1, ssem0, ssem1):
    wid = lax.axis_index("s") * _NC + lax.axis_index("c")
    base_w = wid * _B_PER_W
    rows = [rows0, rows1]
    gsems = [gsem0, gsem1]
    ssems = [ssem0, ssem1]

    pltpu.sync_copy(seq_hbm.at[pl.ds(base_w, _B_PER_W)], idx_v)

    gcopy = [None] * _NCHUNK
    scopy = [None] * _NCHUNK
    gcopy[0] = pltpu.async_copy(
        table_hbm.at[idx_v.at[pl.ds(0, _CHUNK)]], rows[0], gsems[0])
    for i in range(_NCHUNK):
        b = i % 2
        if i + 1 < _NCHUNK:
            nb = (i + 1) % 2
            if i >= 1:
                scopy[i - 1].wait()
            gcopy[i + 1] = pltpu.async_copy(
                table_hbm.at[idx_v.at[pl.ds((i + 1) * _CHUNK, _CHUNK)]],
                rows[nb], gsems[nb])
        gcopy[i].wait()
        scopy[i] = pltpu.async_copy(
            rows[b], out_hbm.at[pl.ds(base_w + i * _CHUNK, _CHUNK)], ssems[b])
    scopy[_NCHUNK - 2].wait()
    scopy[_NCHUNK - 1].wait()


def kernel(seq, W):
    flat = seq.reshape(-1).astype(jnp.int32)
    wtail = jnp.pad(W[_NFULL * _TCH:], ((0, 0), (0, _DP - _D)))
    wp = _transpose(W.T, wtail)
    out = _gather(flat, wp)
    return out[:, :_D].reshape(seq.shape + (_D,))


# speedup vs baseline: 1.1525x; 1.1525x over previous
import functools

import jax
import jax.numpy as jnp
from jax import lax
from jax.experimental import pallas as pl
from jax.experimental.pallas import tpu as pltpu
from jax.experimental.pallas import tpu_sc as plsc

_V = 1000000
_D = 64
_DP = 128
_B = 4096 * 50
_NC = 2
_NS = 16
_NW = _NC * _NS
_B_PER_W = _B // _NW
_CHUNK = 400
_NCHUNK = _B_PER_W // _CHUNK

_TCH = 128
_NFULL = _V // _TCH

_mesh = plsc.VectorSubcoreMesh(core_axis_name="c", subcore_axis_name="s")


@functools.partial(
    pl.kernel,
    out_type=jax.ShapeDtypeStruct((_V, _DP), jnp.float32),
    mesh=_mesh,
    scratch_types=[
        pltpu.VMEM((_D, _TCH), jnp.float32),
        pltpu.VMEM((_D, _TCH), jnp.float32),
        pltpu.VMEM((_TCH, _DP), jnp.float32),
        pltpu.VMEM((_TCH, _DP), jnp.float32),
        pltpu.SemaphoreType.DMA,
        pltpu.SemaphoreType.DMA,
        pltpu.SemaphoreType.DMA,
        pltpu.SemaphoreType.DMA,
    ],
    compiler_params=pltpu.CompilerParams(needs_layout_passes=False),
)
def _transpose(wt_hbm, wtail_hbm, out_hbm, s0, s1, t0, t1, ls0, ls1, ss0, ss1):
    wid = lax.axis_index("s") * _NC + lax.axis_index("c")
    n_mine = (_NFULL - wid + _NW - 1) // _NW
    lane16 = lax.iota(jnp.int32, 16)
    zero16 = jnp.zeros((16,), jnp.int32)

    def load(i, sbuf, lsem):
        base = (wid + i * _NW) * _TCH
        pltpu.async_copy(wt_hbm.at[:, pl.ds(base, _TCH)], sbuf, lsem)

    def load_wait(sbuf, lsem):
        pltpu.make_async_copy(wt_hbm.at[:, pl.ds(0, _TCH)], sbuf, lsem).wait()

    def store(i, tbuf, ssem):
        base = (wid + i * _NW) * _TCH
        pltpu.async_copy(tbuf, out_hbm.at[pl.ds(base, _TCH)], ssem)

    def store_wait(tbuf, ssem):
        pltpu.make_async_copy(
            tbuf, out_hbm.at[pl.ds(0, _TCH)], ssem).wait()

    e_vecs = [lane16 + a * 16 for a in range(4)]

    def transpose_slab(src, dst, nrows):
        def row(d, carry):
            dsplat = zero16 + d
            vals = [plsc.load_gather(src, [e_vecs[a], dsplat])
                    for a in range(4)]
            for a in range(4):
                plsc.store_scatter(dst, [dsplat, e_vecs[a]], vals[a])
            return carry
        lax.fori_loop(0, nrows, row, 0, unroll=8)

    load(0, s0, ls0)

    def pair(p, carry):
        i0 = 2 * p
        i1 = i0 + 1

        @pl.when(i0 < n_mine)
        def _():
            @pl.when(i1 < n_mine)
            def _():
                load(i1, s1, ls1)
            load_wait(s0, ls0)
            transpose_slab(s0, t0, _TCH)

            @pl.when(i0 >= 2)
            def _():
                store_wait(t0, ss0)
            store(i0, t0, ss0)

        @pl.when(i1 < n_mine)
        def _():
            @pl.when(i1 + 1 < n_mine)
            def _():
                load(i1 + 1, s0, ls0)
            load_wait(s1, ls1)
            transpose_slab(s1, t1, _TCH)

            @pl.when(i1 >= 2)
            def _():
                store_wait(t1, ss1)
            store(i1, t1, ss1)
        return carry

    lax.fori_loop(0, (n_mine + 1) // 2, pair, 0)

    store_wait(t0, ss0)
    store_wait(t1, ss1)

    @pl.when(wid == _NFULL % _NW)
    def _():
        pltpu.sync_copy(wtail_hbm, t0.at[pl.ds(0, _D)])
        pltpu.sync_copy(t0.at[pl.ds(0, _D)],
                        out_hbm.at[pl.ds(_NFULL * _TCH, _D)])


@functools.partial(
    pl.kernel,
    out_type=jax.ShapeDtypeStruct((_B, _DP), jnp.float32),
    mesh=_mesh,
    scratch_types=[
        pltpu.VMEM((_B_PER_W,), jnp.int32),
        pltpu.VMEM((_CHUNK, _DP), jnp.float32),
        pltpu.VMEM((_CHUNK, _DP), jnp.float32),
        pltpu.SemaphoreType.DMA,
        pltpu.SemaphoreType.DMA,
        pltpu.SemaphoreType.DMA,
        pltpu.SemaphoreType.DMA,
    ],
)
def _gather(seq_hbm, table_hbm, out_hbm, idx_v, rows0, rows1,
            gsem0, gsem1, ssem0, ssem1):
    wid = lax.axis_index("s") * _NC + lax.axis_index("c")
    base_w = wid * _B_PER_W
    rows = [rows0, rows1]
    gsems = [gsem0, gsem1]
    ssems = [ssem0, ssem1]

    pltpu.sync_copy(seq_hbm.at[pl.ds(base_w, _B_PER_W)], idx_v)

    gcopy = [None] * _NCHUNK
    scopy = [None] * _NCHUNK
    gcopy[0] = pltpu.async_copy(
        table_hbm.at[idx_v.at[pl.ds(0, _CHUNK)]], rows[0], gsems[0])
    for i in range(_NCHUNK):
        b = i % 2
        if i + 1 < _NCHUNK:
            nb = (i + 1) % 2
            if i >= 1:
                scopy[i - 1].wait()
            gcopy[i + 1] = pltpu.async_copy(
                table_hbm.at[idx_v.at[pl.ds((i + 1) * _CHUNK, _CHUNK)]],
                rows[nb], gsems[nb])
        gcopy[i].wait()
        scopy[i] = pltpu.async_copy(
            rows[b], out_hbm.at[pl.ds(base_w + i * _CHUNK, _CHUNK)], ssems[b])
    scopy[_NCHUNK - 2].wait()
    scopy[_NCHUNK - 1].wait()


def kernel(seq, W):
    flat = seq.reshape(-1).astype(jnp.int32)
    wtail = jnp.pad(W[_NFULL * _TCH:], ((0, 0), (0, _DP - _D)))
    wp = _transpose(W.T, wtail)
    out = _gather(flat, wp)
    return out[:, :_D].reshape(seq.shape + (_D,))


# speedup vs baseline: 3.2946x; 2.8588x over previous
import functools

import jax
import jax.numpy as jnp
from jax import lax
from jax.experimental import pallas as pl
from jax.experimental.pallas import tpu as pltpu
from jax.experimental.pallas import tpu_sc as plsc

_V = 1000000
_D = 64
_DP = 128
_B = 4096 * 50
_NC = 2
_NS = 16
_NW = _NC * _NS
_B_PER_W = _B // _NW
_CHUNK = 400
_NCHUNK = _B_PER_W // _CHUNK

_TCH = 128
_NFULL = _V // _TCH

_mesh = plsc.VectorSubcoreMesh(core_axis_name="c", subcore_axis_name="s")


@functools.partial(
    pl.kernel,
    out_type=jax.ShapeDtypeStruct((_V, _DP), jnp.float32),
    mesh=_mesh,
    scratch_types=[
        pltpu.VMEM((_D, _TCH), jnp.float32),
        pltpu.VMEM((_D, _TCH), jnp.float32),
        pltpu.VMEM((_TCH, _DP), jnp.float32),
        pltpu.VMEM((_TCH, _DP), jnp.float32),
        pltpu.SemaphoreType.DMA,
        pltpu.SemaphoreType.DMA,
        pltpu.SemaphoreType.DMA,
        pltpu.SemaphoreType.DMA,
    ],
    compiler_params=pltpu.CompilerParams(needs_layout_passes=False),
)
def _transpose(wt_hbm, wtail_hbm, out_hbm, s0, s1, t0, t1, ls0, ls1, ss0, ss1):
    wid = lax.axis_index("s") * _NC + lax.axis_index("c")
    n_mine = (_NFULL - wid + _NW - 1) // _NW
    lane16 = lax.iota(jnp.int32, 16)
    zero16 = jnp.zeros((16,), jnp.int32)

    def load(i, sbuf, lsem):
        base = (wid + i * _NW) * _TCH
        pltpu.async_copy(wt_hbm.at[:, pl.ds(base, _TCH)], sbuf, lsem)

    def load_wait(sbuf, lsem):
        pltpu.make_async_copy(wt_hbm.at[:, pl.ds(0, _TCH)], sbuf, lsem).wait()

    def store(i, tbuf, ssem):
        base = (wid + i * _NW) * _TCH
        pltpu.async_copy(tbuf, out_hbm.at[pl.ds(base, _TCH)], ssem)

    def store_wait(tbuf, ssem):
        pltpu.make_async_copy(
            tbuf, out_hbm.at[pl.ds(0, _TCH)], ssem).wait()

    e_vecs = [lane16 + a * 16 for a in range(4)]

    def transpose_slab(src, dst, nrows):
        @functools.partial(plsc.parallel_loop, 0, nrows, unroll=8)
        def row(d):
            dsplat = zero16 + d
            vals = [plsc.load_gather(src, [e_vecs[a], dsplat])
                    for a in range(4)]
            for a in range(4):
                plsc.store_scatter(dst, [dsplat, e_vecs[a]], vals[a])

    load(0, s0, ls0)

    def pair(p, carry):
        i0 = 2 * p
        i1 = i0 + 1

        @pl.when(i0 < n_mine)
        def _():
            @pl.when(i1 < n_mine)
            def _():
                load(i1, s1, ls1)
            load_wait(s0, ls0)
            transpose_slab(s0, t0, _TCH)

            @pl.when(i0 >= 2)
            def _():
                store_wait(t0, ss0)
            store(i0, t0, ss0)

        @pl.when(i1 < n_mine)
        def _():
            @pl.when(i1 + 1 < n_mine)
            def _():
                load(i1 + 1, s0, ls0)
            load_wait(s1, ls1)
            transpose_slab(s1, t1, _TCH)

            @pl.when(i1 >= 2)
            def _():
                store_wait(t1, ss1)
            store(i1, t1, ss1)
        return carry

    lax.fori_loop(0, (n_mine + 1) // 2, pair, 0)

    store_wait(t0, ss0)
    store_wait(t1, ss1)

    @pl.when(wid == _NFULL % _NW)
    def _():
        pltpu.sync_copy(wtail_hbm, t0.at[pl.ds(0, _D)])
        pltpu.sync_copy(t0.at[pl.ds(0, _D)],
                        out_hbm.at[pl.ds(_NFULL * _TCH, _D)])


@functools.partial(
    pl.kernel,
    out_type=jax.ShapeDtypeStruct((_B, _DP), jnp.float32),
    mesh=_mesh,
    scratch_types=[
        pltpu.VMEM((_B_PER_W,), jnp.int32),
        pltpu.VMEM((_CHUNK, _DP), jnp.float32),
        pltpu.VMEM((_CHUNK, _DP), jnp.float32),
        pltpu.SemaphoreType.DMA,
        pltpu.SemaphoreType.DMA,
        pltpu.SemaphoreType.DMA,
        pltpu.SemaphoreType.DMA,
    ],
)
def _gather(seq_hbm, table_hbm, out_hbm, idx_v, rows0, rows1,
            gsem0, gsem1, ssem0, ssem1):
    wid = lax.axis_index("s") * _NC + lax.axis_index("c")
    base_w = wid * _B_PER_W
    rows = [rows0, rows1]
    gsems = [gsem0, gsem1]
    ssems = [ssem0, ssem1]

    pltpu.sync_copy(seq_hbm.at[pl.ds(base_w, _B_PER_W)], idx_v)

    gcopy = [None] * _NCHUNK
    scopy = [None] * _NCHUNK
    gcopy[0] = pltpu.async_copy(
        table_hbm.at[idx_v.at[pl.ds(0, _CHUNK)]], rows[0], gsems[0])
    for i in range(_NCHUNK):
        b = i % 2
        if i + 1 < _NCHUNK:
            nb = (i + 1) % 2
            if i >= 1:
                scopy[i - 1].wait()
            gcopy[i + 1] = pltpu.async_copy(
                table_hbm.at[idx_v.at[pl.ds((i + 1) * _CHUNK, _CHUNK)]],
                rows[nb], gsems[nb])
        gcopy[i].wait()
        scopy[i] = pltpu.async_copy(
            rows[b], out_hbm.at[pl.ds(base_w + i * _CHUNK, _CHUNK)], ssems[b])
    scopy[_NCHUNK - 2].wait()
    scopy[_NCHUNK - 1].wait()


def kernel(seq, W):
    flat = seq.reshape(-1).astype(jnp.int32)
    wtail = jnp.pad(W[_NFULL * _TCH:], ((0, 0), (0, _DP - _D)))
    wp = _transpose(W.T, wtail)
    out = _gather(flat, wp)
    return out[:, :_D].reshape(seq.shape + (_D,))


# speedup vs baseline: 4.1028x; 1.2453x over previous
import functools

import jax
import jax.numpy as jnp
from jax import lax
from jax.experimental import pallas as pl
from jax.experimental.pallas import tpu as pltpu
from jax.experimental.pallas import tpu_sc as plsc

_V = 1000000
_D = 64
_DP = 128
_B = 4096 * 50
_NC = 2
_NS = 16
_NW = _NC * _NS
_B_PER_W = _B // _NW
_CHUNK = 400
_NCHUNK = _B_PER_W // _CHUNK

_TCH = 128
_NFULL = _V // _TCH

_mesh = plsc.VectorSubcoreMesh(core_axis_name="c", subcore_axis_name="s")


@functools.partial(
    pl.kernel,
    out_type=jax.ShapeDtypeStruct((_V, _DP), jnp.float32),
    mesh=_mesh,
    scratch_types=[
        pltpu.VMEM((_D, _TCH), jnp.float32),
        pltpu.VMEM((_D, _TCH), jnp.float32),
        pltpu.VMEM((_TCH, _DP), jnp.float32),
        pltpu.VMEM((_TCH, _DP), jnp.float32),
        pltpu.SemaphoreType.DMA,
        pltpu.SemaphoreType.DMA,
        pltpu.SemaphoreType.DMA,
        pltpu.SemaphoreType.DMA,
    ],
    compiler_params=pltpu.CompilerParams(needs_layout_passes=False),
)
def _transpose(wt_hbm, wtail_hbm, out_hbm, s0, s1, t0, t1, ls0, ls1, ss0, ss1):
    wid = lax.axis_index("s") * _NC + lax.axis_index("c")
    n_mine = (_NFULL - wid + _NW - 1) // _NW
    lane16 = lax.iota(jnp.int32, 16)
    zero16 = jnp.zeros((16,), jnp.int32)

    def load(i, sbuf, lsem):
        base = (wid + i * _NW) * _TCH
        pltpu.async_copy(wt_hbm.at[:, pl.ds(base, _TCH)], sbuf, lsem)

    def load_wait(sbuf, lsem):
        pltpu.make_async_copy(wt_hbm.at[:, pl.ds(0, _TCH)], sbuf, lsem).wait()

    def store(i, tbuf, ssem):
        base = (wid + i * _NW) * _TCH
        pltpu.async_copy(tbuf, out_hbm.at[pl.ds(base, _TCH)], ssem)

    def store_wait(tbuf, ssem):
        pltpu.make_async_copy(
            tbuf, out_hbm.at[pl.ds(0, _TCH)], ssem).wait()

    e_vecs = [lane16 + a * 16 for a in range(4)]

    def transpose_slab(src, dst, nrows):
        @functools.partial(plsc.parallel_loop, 0, nrows, unroll=8)
        def row(d):
            dsplat = zero16 + d
            vals = [plsc.load_gather(src, [e_vecs[a], dsplat])
                    for a in range(4)]
            for a in range(4):
                plsc.store_scatter(dst, [dsplat, e_vecs[a]], vals[a])

    load(0, s0, ls0)

    def pair(p, carry):
        i0 = 2 * p
        i1 = i0 + 1

        @pl.when(i0 < n_mine)
        def _():
            @pl.when(i1 < n_mine)
            def _():
                load(i1, s1, ls1)
            load_wait(s0, ls0)
            transpose_slab(s0, t0, _TCH)

            @pl.when(i0 >= 2)
            def _():
                store_wait(t0, ss0)
            store(i0, t0, ss0)

        @pl.when(i1 < n_mine)
        def _():
            @pl.when(i1 + 1 < n_mine)
            def _():
                load(i1 + 1, s0, ls0)
            load_wait(s1, ls1)
            transpose_slab(s1, t1, _TCH)

            @pl.when(i1 >= 2)
            def _():
                store_wait(t1, ss1)
            store(i1, t1, ss1)
        return carry

    lax.fori_loop(0, (n_mine + 1) // 2, pair, 0)

    store_wait(t0, ss0)
    store_wait(t1, ss1)

    @pl.when(wid == _NFULL % _NW)
    def _():
        pltpu.sync_copy(wtail_hbm, t0.at[pl.ds(0, _D)])
        pltpu.sync_copy(t0.at[pl.ds(0, _D)],
                        out_hbm.at[pl.ds(_NFULL * _TCH, _D)])


_T = 50
_TP = 56
_BATCH = 4096
_BP = _BATCH * _TP


@functools.partial(
    pl.kernel,
    out_type=jax.ShapeDtypeStruct((_BP, _DP), jnp.float32),
    mesh=_mesh,
    scratch_types=[
        pltpu.VMEM((_B_PER_W,), jnp.int32),
        pltpu.VMEM((_CHUNK,), jnp.int32),
        pltpu.VMEM((_CHUNK,), jnp.int32),
        pltpu.VMEM((_CHUNK, _DP), jnp.float32),
        pltpu.VMEM((_CHUNK, _DP), jnp.float32),
        pltpu.SemaphoreType.DMA,
        pltpu.SemaphoreType.DMA,
        pltpu.SemaphoreType.DMA,
        pltpu.SemaphoreType.DMA,
        pltpu.SemaphoreType.DMA,
        pltpu.SemaphoreType.DMA,
    ],
)
def _gather(seq_hbm, opos_hbm, table_hbm, out_hbm, idx_v, op0, op1,
            rows0, rows1, gsem0, gsem1, ssem0, ssem1, osem0, osem1):
    wid = lax.axis_index("s") * _NC + lax.axis_index("c")
    base_w = wid * _B_PER_W
    rows = [rows0, rows1]
    opos = [op0, op1]
    gsems = [gsem0, gsem1]
    ssems = [ssem0, ssem1]
    osems = [osem0, osem1]

    pltpu.sync_copy(seq_hbm.at[pl.ds(base_w, _B_PER_W)], idx_v)

    gcopy = [None] * _NCHUNK
    scopy = [None] * _NCHUNK
    ocopy = [None] * _NCHUNK
    gcopy[0] = pltpu.async_copy(
        table_hbm.at[idx_v.at[pl.ds(0, _CHUNK)]], rows[0], gsems[0])
    ocopy[0] = pltpu.async_copy(
        opos_hbm.at[pl.ds(base_w, _CHUNK)], opos[0], osems[0])
    for i in range(_NCHUNK):
        b = i % 2
        if i + 1 < _NCHUNK:
            nb = (i + 1) % 2
            if i >= 1:
                scopy[i - 1].wait()
            gcopy[i + 1] = pltpu.async_copy(
                table_hbm.at[idx_v.at[pl.ds((i + 1) * _CHUNK, _CHUNK)]],
                rows[nb], gsems[nb])
            ocopy[i + 1] = pltpu.async_copy(
                opos_hbm.at[pl.ds(base_w + (i + 1) * _CHUNK, _CHUNK)],
                opos[nb], osems[nb])
        gcopy[i].wait()
        ocopy[i].wait()
        scopy[i] = pltpu.async_copy(
            rows[b], out_hbm.at[opos[b]], ssems[b])
    scopy[_NCHUNK - 2].wait()
    scopy[_NCHUNK - 1].wait()


def kernel(seq, W):
    flat = seq.reshape(-1).astype(jnp.int32)
    pos = jnp.arange(_B, dtype=jnp.int32)
    opos = (pos // _T) * _TP + pos % _T
    wtail = jnp.pad(W[_NFULL * _TCH:], ((0, 0), (0, _DP - _D)))
    wp = _transpose(W.T, wtail)
    out = _gather(flat, opos, wp)
    return out[:, :_D].reshape(_BATCH, _TP, _D)[:, :_T, :]
